# baseline (speedup 1.0000x reference)
import functools

import jax
import jax.numpy as jnp
from jax import lax
from jax.experimental import pallas as pl
from jax.experimental.pallas import tpu as pltpu
from jax.experimental.pallas import tpu_sc as plsc

N = 10000
E = 320000
D = 128
H = 128
B = 512
FFN = 300
OUT = 1
DEPTH = 3

NC = 2
NS = 16
CH = 128
GH = CH // 4
NCHUNK = 160
NHALF = 40
EPW = NCHUNK * CH
E_PAD = NS * EPW
NP = 10240
RPS = NP // NS


def _sc_edge_agg(h2, src4, dst4, zrows):
    mesh = plsc.VectorSubcoreMesh(core_axis_name="c", subcore_axis_name="s")

    @functools.partial(
        pl.kernel,
        out_type=jax.ShapeDtypeStruct((NC, NP, H), jnp.float32),
        mesh=mesh,
        scratch_types=[
            pltpu.VMEM((NHALF, CH), jnp.int32),
            pltpu.VMEM((NHALF, CH), jnp.int32),
            pltpu.VMEM((2 * CH, H), jnp.float32),
            pltpu.VMEM_SHARED((NP, H), jnp.float32),
            [pltpu.SemaphoreType.DMA] * 8,
        ],
    )
    def k(h_hbm, src_hbm, dst_hbm, z_hbm, out_hbm, src_v, dst_v, rows,
          agg_s, sems):
        c = lax.axis_index("c")
        s = lax.axis_index("s")
        pltpu.sync_copy(z_hbm, agg_s.at[pl.ds(s * RPS, RPS)])
        plsc.subcore_barrier()

        htab = h_hbm.at[c]

        def gslot(p, half):
            return rows.at[pl.ds(p * CH + half * GH, GH)]

        def sbuf(p):
            return rows.at[pl.ds(p * CH, CH)]

        def start_gathers(p, j):
            for half in range(4):
                pltpu.async_copy(
                    htab.at[src_v.at[j, pl.ds(half * GH, GH)]],
                    gslot(p, half), sems[4 * p + half])

        def wait_scatter(p, j):
            for half in range(4):
                pltpu.make_async_copy(
                    htab.at[src_v.at[0, pl.ds(0, GH)]],
                    gslot(p, half), sems[4 * p + half]).wait()
            pltpu.sync_copy(sbuf(p), agg_s.at[dst_v.at[j]], add=True)

        for hh in range(NCHUNK // NHALF):
            pltpu.sync_copy(src_hbm.at[c, s, pl.ds(hh * NHALF, NHALF)], src_v)
            pltpu.sync_copy(dst_hbm.at[c, s, pl.ds(hh * NHALF, NHALF)], dst_v)
            start_gathers(0, 0)
            start_gathers(1, 1)

            def body(q, carry):
                j = 2 * q
                for p in range(2):
                    wait_scatter(p, j + p)
                    start_gathers(p, j + p + 2)
                return carry

            lax.fori_loop(0, NHALF // 2 - 1, body, 0, unroll=False)
            for p in range(2):
                wait_scatter(p, NHALF - 2 + p)
        plsc.subcore_barrier()
        pltpu.sync_copy(agg_s.at[pl.ds(s * RPS, RPS)],
                        out_hbm.at[c, pl.ds(s * RPS, RPS)])

    return k(h2, src4, dst4, zrows)


def _relu(x):
    return jnp.maximum(x, 0.0)


def _dot(a, b):
    return jnp.dot(a, b, preferred_element_type=jnp.float32)


def _h0_body(xl_ref, xr_ref, wl_ref, wr_ref, o_ref):
    o_ref[0] = _relu(_dot(xl_ref[...], wl_ref[...]))
    o_ref[1] = _relu(_dot(xr_ref[...], wr_ref[...]))


def _h0(xl, xr, wl, wr):
    return pl.pallas_call(
        _h0_body,
        out_shape=jax.ShapeDtypeStruct((NC, N, H), jnp.float32))(xl, xr, wl, wr)


def _step_body(agg_ref, h0_ref, wl_ref, wr_ref, o_ref):
    o_ref[0] = _relu(h0_ref[0] + _dot(agg_ref[0, :N, :], wl_ref[...]))
    o_ref[1] = _relu(h0_ref[1] + _dot(agg_ref[1, :N, :], wr_ref[...]))


def _step(agg, h0, wl, wr):
    return pl.pallas_call(
        _step_body,
        out_shape=jax.ShapeDtypeStruct((NC, N, H), jnp.float32))(agg, h0, wl, wr)


def _readout_body(xl_ref, xr_ref, h_ref, wl_ref, wr_ref, l_ref, r_ref):
    wl = wl_ref[...]
    wr = wr_ref[...]
    l_ref[...] = _relu(_dot(xl_ref[...], wl[:D]) + _dot(h_ref[0], wl[D:]))
    r_ref[...] = _relu(_dot(xr_ref[...], wr[:D]) + _dot(h_ref[1], wr[D:]))


def _readout(xl, xr, h, wl, wr):
    return pl.pallas_call(
        _readout_body,
        out_shape=(jax.ShapeDtypeStruct((N, H), jnp.float32),
                   jax.ShapeDtypeStruct((N, H), jnp.float32)),
    )(xl, xr, h, wl, wr)


def _pool_body(la_ref, ra_ref, bl_ref, br_ref, lo_ref, ro_ref):
    for b_ref, a_ref, o_ref in ((bl_ref, la_ref, lo_ref),
                                (br_ref, ra_ref, ro_ref)):
        onehot = (b_ref[...][None, :] ==
                  lax.broadcasted_iota(jnp.int32, (B, N), 0)).astype(jnp.float32)
        counts = jnp.sum(onehot, axis=1)
        o_ref[...] = _dot(onehot, a_ref[...]) / jnp.maximum(counts, 1.0)[:, None]


def _pool(la, ra, bl, br):
    return pl.pallas_call(
        _pool_body,
        out_shape=(jax.ShapeDtypeStruct((B, H), jnp.float32),
                   jax.ShapeDtypeStruct((B, H), jnp.float32)),
    )(la, ra, bl, br)


def _coatt_body(atom_ref, batch_ref, other_ref, wi_ref, wib_ref, prj_ref,
                prjb_ref, sc_ref, seg_ref):
    seg = batch_ref[...]
    onehot = (seg[None, :] == lax.broadcasted_iota(jnp.int32, (B, N), 0)
              ).astype(jnp.float32)
    atom = atom_ref[...]
    other = other_ref[...]
    a = _dot(atom, wi_ref[...]) + wib_ref[...][None, :]
    p_other = _dot(other, prj_ref[...]) + prjb_ref[...][None, :]
    dn = (((0,), (0,)), ((), ()))
    align_p = lax.dot_general(onehot, p_other, dn,
                              preferred_element_type=jnp.float32)
    scores = jnp.sum(a * align_p, axis=-1)
    mask = onehot > 0.0
    mx = jnp.max(jnp.where(mask, scores[None, :], -jnp.inf), axis=1)
    mx = jnp.where(jnp.isfinite(mx), mx, 0.0)
    mxg = lax.dot_general(onehot, mx, dn, preferred_element_type=jnp.float32)
    e = jnp.exp(scores - mxg)
    ssum = _dot(onehot, e)
    esg = lax.dot_general(onehot, ssum, dn, preferred_element_type=jnp.float32)
    sm = e / (esg + 1e-16)
    sc_ref[...] = sm
    align = lax.dot_general(onehot, other, dn,
                            preferred_element_type=jnp.float32)
    seg_ref[...] = _dot(onehot, atom * align * sm[:, None])


def _coatt(atom, batch, other_out, wi, wib, prj, prjb):
    return pl.pallas_call(
        _coatt_body,
        out_shape=(jax.ShapeDtypeStruct((N,), jnp.float32),
                   jax.ShapeDtypeStruct((B, H), jnp.float32)),
    )(atom, batch, other_out, wi, wib, prj, prjb)


def _ffn_body(h_ref, t_ref, noise_ref, w1_ref, b1_ref, w2_ref, b2_ref,
              out_ref, hp_ref):
    h = h_ref[...]
    t = t_ref[...]
    nz = noise_ref[...]
    hp = h + jnp.sign(h) * nz * 0.1
    tp = t + jnp.sign(t) * nz * 0.1
    hid = _relu(_dot(hp, w1_ref[0:H]) + _dot(tp, w1_ref[H:]) + b1_ref[...][None, :])
    out_ref[...] = _dot(hid, w2_ref[...]) + b2_ref[...][None, :]
    hp_ref[...] = hp


def _ffn(h_out, t_out, noise, w1, b1, w2, b2):
    return pl.pallas_call(
        _ffn_body,
        out_shape=(jax.ShapeDtypeStruct((B, OUT), jnp.float32),
                   jax.ShapeDtypeStruct((B, H), jnp.float32)),
    )(h_out, t_out, noise, w1, b1, w2, b2)


def _prep_edges(edge_index_left, edge_index_right):
    pad = jnp.arange(E_PAD - E, dtype=jnp.int32)

    def side(edge_index):
        src = jnp.concatenate(
            [edge_index[0], pad % N]).reshape(NS, NCHUNK, CH)
        dst = jnp.concatenate(
            [edge_index[1], N + pad % (NP - N)]).reshape(NS, NCHUNK, CH)
        return src, dst

    sl, dl = side(edge_index_left)
    sr, dr = side(edge_index_right)
    return jnp.stack([sl, sr]), jnp.stack([dl, dr])


def kernel(x_left, edge_index_left, batch_left, x_right, edge_index_right,
           batch_right, W_i1, W_h1, W_o1, W_i2, W_h2, W_o2, w_i_w, w_i_b,
           prj_i_w, prj_i_b, ffn1_w, ffn1_b, ffn2_w, ffn2_b):
    zrows = jnp.zeros((RPS, H), jnp.float32)
    src4, dst4 = _prep_edges(edge_index_left, edge_index_right)

    h0 = _h0(x_left, x_right, W_i1, W_i2)
    h = h0
    for _ in range(DEPTH):
        agg = _sc_edge_agg(h, src4, dst4, zrows)
        h = _step(agg, h0, W_h1, W_h2)
    left_atom, right_atom = _readout(x_left, x_right, h, W_o1, W_o2)

    left_out, right_out = _pool(left_atom, right_atom, batch_left, batch_right)

    left_scores, h_output = _coatt(left_atom, batch_left, right_out,
                                   w_i_w, w_i_b, prj_i_w, prj_i_b)
    right_scores, t_output = _coatt(right_atom, batch_right, left_out,
                                    w_i_w, w_i_b, prj_i_w, prj_i_b)

    noise = jax.random.uniform(jax.random.key(42), (B, H), jnp.float32)
    noise = noise / (jnp.linalg.norm(noise, axis=-1, keepdims=True) + 1e-12)

    output, h_pert = _ffn(h_output, t_output, noise, ffn1_w, ffn1_b,
                          ffn2_w, ffn2_b)
    return (output, h_output, h_pert, left_scores, right_scores,
            left_out, right_out)


# speedup vs baseline: 1.0288x; 1.0288x over previous
import functools

import jax
import jax.numpy as jnp
from jax import lax
from jax.experimental import pallas as pl
from jax.experimental.pallas import tpu as pltpu
from jax.experimental.pallas import tpu_sc as plsc

N = 10000
E = 320000
D = 128
H = 128
B = 512
FFN = 300
OUT = 1
DEPTH = 3

NC = 2
NS = 16
CH = 128
GH = CH // 2
NCHUNK = 160
NHALF = 40
EPW = NCHUNK * CH
E_PAD = NS * EPW
NP = 10240
RPS = NP // NS


def _sc_edge_agg(h2, src4, dst4, zrows):
    mesh = plsc.VectorSubcoreMesh(core_axis_name="c", subcore_axis_name="s")

    @functools.partial(
        pl.kernel,
        out_type=jax.ShapeDtypeStruct((NC, NP, H), jnp.float32),
        mesh=mesh,
        scratch_types=[
            pltpu.VMEM((NHALF, CH), jnp.int32),
            pltpu.VMEM((NHALF, CH), jnp.int32),
            pltpu.VMEM((2 * CH, H), jnp.float32),
            pltpu.VMEM_SHARED((NP, H), jnp.float32),
            [pltpu.SemaphoreType.DMA] * 4,
        ],
    )
    def k(h_hbm, src_hbm, dst_hbm, z_hbm, out_hbm, src_v, dst_v, rows,
          agg_s, sems):
        c = lax.axis_index("c")
        s = lax.axis_index("s")
        pltpu.sync_copy(z_hbm, agg_s.at[pl.ds(s * RPS, RPS)])
        plsc.subcore_barrier()

        htab = h_hbm.at[c]

        def gslot(p, half):
            return rows.at[pl.ds(p * CH + half * GH, GH)]

        def sbuf(p):
            return rows.at[pl.ds(p * CH, CH)]

        def start_gathers(p, j):
            for half in range(2):
                pltpu.async_copy(
                    htab.at[src_v.at[j, pl.ds(half * GH, GH)]],
                    gslot(p, half), sems[2 * p + half])

        def wait_scatter(p, j):
            for half in range(2):
                pltpu.make_async_copy(
                    htab.at[src_v.at[0, pl.ds(0, GH)]],
                    gslot(p, half), sems[2 * p + half]).wait()
            pltpu.sync_copy(sbuf(p), agg_s.at[dst_v.at[j]], add=True)

        for hh in range(NCHUNK // NHALF):
            pltpu.sync_copy(src_hbm.at[c, s, pl.ds(hh * NHALF, NHALF)], src_v)
            pltpu.sync_copy(dst_hbm.at[c, s, pl.ds(hh * NHALF, NHALF)], dst_v)
            start_gathers(0, 0)
            start_gathers(1, 1)

            def body(q, carry):
                j = 2 * q
                for p in range(2):
                    wait_scatter(p, j + p)
                    start_gathers(p, j + p + 2)
                return carry

            lax.fori_loop(0, NHALF // 2 - 1, body, 0, unroll=False)
            for p in range(2):
                wait_scatter(p, NHALF - 2 + p)
        plsc.subcore_barrier()
        pltpu.sync_copy(agg_s.at[pl.ds(s * RPS, RPS)],
                        out_hbm.at[c, pl.ds(s * RPS, RPS)])

    return k(h2, src4, dst4, zrows)


def _relu(x):
    return jnp.maximum(x, 0.0)


def _dot(a, b):
    return jnp.dot(a, b, preferred_element_type=jnp.float32)


def _h0_body(xl_ref, xr_ref, wl_ref, wr_ref, o_ref):
    o_ref[0] = _relu(_dot(xl_ref[...], wl_ref[...]))
    o_ref[1] = _relu(_dot(xr_ref[...], wr_ref[...]))


def _h0(xl, xr, wl, wr):
    return pl.pallas_call(
        _h0_body,
        out_shape=jax.ShapeDtypeStruct((NC, N, H), jnp.float32))(xl, xr, wl, wr)


def _step_body(agg_ref, h0_ref, wl_ref, wr_ref, o_ref):
    o_ref[0] = _relu(h0_ref[0] + _dot(agg_ref[0, :N, :], wl_ref[...]))
    o_ref[1] = _relu(h0_ref[1] + _dot(agg_ref[1, :N, :], wr_ref[...]))


def _step(agg, h0, wl, wr):
    return pl.pallas_call(
        _step_body,
        out_shape=jax.ShapeDtypeStruct((NC, N, H), jnp.float32))(agg, h0, wl, wr)


def _last_body(agg_ref, h0_ref, wl_ref, wr_ref, xl_ref, xr_ref,
               wol_ref, wor_ref, l_ref, r_ref):
    hl = _relu(h0_ref[0] + _dot(agg_ref[0, :N, :], wl_ref[...]))
    hr = _relu(h0_ref[1] + _dot(agg_ref[1, :N, :], wr_ref[...]))
    wol = wol_ref[...]
    wor = wor_ref[...]
    l_ref[...] = _relu(_dot(xl_ref[...], wol[:D]) + _dot(hl, wol[D:]))
    r_ref[...] = _relu(_dot(xr_ref[...], wor[:D]) + _dot(hr, wor[D:]))


def _last_step(agg, h0, wl, wr, xl, xr, wol, wor):
    return pl.pallas_call(
        _last_body,
        out_shape=(jax.ShapeDtypeStruct((N, H), jnp.float32),
                   jax.ShapeDtypeStruct((N, H), jnp.float32)),
    )(agg, h0, wl, wr, xl, xr, wol, wor)


def _pool_body(la_ref, ra_ref, bl_ref, br_ref, lo_ref, ro_ref):
    for b_ref, a_ref, o_ref in ((bl_ref, la_ref, lo_ref),
                                (br_ref, ra_ref, ro_ref)):
        onehot = (b_ref[...][None, :] ==
                  lax.broadcasted_iota(jnp.int32, (B, N), 0)).astype(jnp.float32)
        counts = jnp.sum(onehot, axis=1)
        o_ref[...] = _dot(onehot, a_ref[...]) / jnp.maximum(counts, 1.0)[:, None]


def _pool(la, ra, bl, br):
    return pl.pallas_call(
        _pool_body,
        out_shape=(jax.ShapeDtypeStruct((B, H), jnp.float32),
                   jax.ShapeDtypeStruct((B, H), jnp.float32)),
    )(la, ra, bl, br)


def _coatt_body(atom_ref, batch_ref, other_ref, wi_ref, wib_ref, prj_ref,
                prjb_ref, sc_ref, seg_ref):
    seg = batch_ref[...]
    onehot = (seg[None, :] == lax.broadcasted_iota(jnp.int32, (B, N), 0)
              ).astype(jnp.float32)
    atom = atom_ref[...]
    other = other_ref[...]
    a = _dot(atom, wi_ref[...]) + wib_ref[...][None, :]
    p_other = _dot(other, prj_ref[...]) + prjb_ref[...][None, :]
    dn = (((0,), (0,)), ((), ()))
    align_p = lax.dot_general(onehot, p_other, dn,
                              preferred_element_type=jnp.float32)
    scores = jnp.sum(a * align_p, axis=-1)
    mask = onehot > 0.0
    mx = jnp.max(jnp.where(mask, scores[None, :], -jnp.inf), axis=1)
    mx = jnp.where(jnp.isfinite(mx), mx, 0.0)
    mxg = lax.dot_general(onehot, mx, dn, preferred_element_type=jnp.float32)
    e = jnp.exp(scores - mxg)
    ssum = _dot(onehot, e)
    esg = lax.dot_general(onehot, ssum, dn, preferred_element_type=jnp.float32)
    sm = e / (esg + 1e-16)
    sc_ref[...] = sm
    align = lax.dot_general(onehot, other, dn,
                            preferred_element_type=jnp.float32)
    seg_ref[...] = _dot(onehot, atom * align * sm[:, None])


def _coatt(atom, batch, other_out, wi, wib, prj, prjb):
    return pl.pallas_call(
        _coatt_body,
        out_shape=(jax.ShapeDtypeStruct((N,), jnp.float32),
                   jax.ShapeDtypeStruct((B, H), jnp.float32)),
    )(atom, batch, other_out, wi, wib, prj, prjb)


def _coatt_ffn_body(atom_ref, batch_ref, other_ref, wi_ref, wib_ref, prj_ref,
                    prjb_ref, hout_ref, noise_ref, w1_ref, b1_ref, w2_ref,
                    b2_ref, sc_ref, out_ref, hp_ref):
    seg = batch_ref[...]
    onehot = (seg[None, :] == lax.broadcasted_iota(jnp.int32, (B, N), 0)
              ).astype(jnp.float32)
    atom = atom_ref[...]
    other = other_ref[...]
    a = _dot(atom, wi_ref[...]) + wib_ref[...][None, :]
    p_other = _dot(other, prj_ref[...]) + prjb_ref[...][None, :]
    dn = (((0,), (0,)), ((), ()))
    align_p = lax.dot_general(onehot, p_other, dn,
                              preferred_element_type=jnp.float32)
    scores = jnp.sum(a * align_p, axis=-1)
    mask = onehot > 0.0
    mx = jnp.max(jnp.where(mask, scores[None, :], -jnp.inf), axis=1)
    mx = jnp.where(jnp.isfinite(mx), mx, 0.0)
    mxg = lax.dot_general(onehot, mx, dn, preferred_element_type=jnp.float32)
    e = jnp.exp(scores - mxg)
    ssum = _dot(onehot, e)
    esg = lax.dot_general(onehot, ssum, dn, preferred_element_type=jnp.float32)
    sm = e / (esg + 1e-16)
    sc_ref[...] = sm
    align = lax.dot_general(onehot, other, dn,
                            preferred_element_type=jnp.float32)
    t_out = _dot(onehot, atom * align * sm[:, None])
    h = hout_ref[...]
    nz = noise_ref[...]
    hp = h + jnp.sign(h) * nz * 0.1
    tp = t_out + jnp.sign(t_out) * nz * 0.1
    hid = _relu(_dot(hp, w1_ref[0:H]) + _dot(tp, w1_ref[H:])
                + b1_ref[...][None, :])
    out_ref[...] = _dot(hid, w2_ref[...]) + b2_ref[...][None, :]
    hp_ref[...] = hp


def _coatt_ffn(atom, batch, other_out, wi, wib, prj, prjb, h_out, noise,
               w1, b1, w2, b2):
    return pl.pallas_call(
        _coatt_ffn_body,
        out_shape=(jax.ShapeDtypeStruct((N,), jnp.float32),
                   jax.ShapeDtypeStruct((B, OUT), jnp.float32),
                   jax.ShapeDtypeStruct((B, H), jnp.float32)),
    )(atom, batch, other_out, wi, wib, prj, prjb, h_out, noise, w1, b1, w2, b2)


def _prep_edges(edge_index_left, edge_index_right):
    pad = jnp.arange(E_PAD - E, dtype=jnp.int32)

    def side(edge_index):
        src = jnp.concatenate(
            [edge_index[0], pad % N]).reshape(NS, NCHUNK, CH)
        dst = jnp.concatenate(
            [edge_index[1], N + pad % (NP - N)]).reshape(NS, NCHUNK, CH)
        return src, dst

    sl, dl = side(edge_index_left)
    sr, dr = side(edge_index_right)
    return jnp.stack([sl, sr]), jnp.stack([dl, dr])


def kernel(x_left, edge_index_left, batch_left, x_right, edge_index_right,
           batch_right, W_i1, W_h1, W_o1, W_i2, W_h2, W_o2, w_i_w, w_i_b,
           prj_i_w, prj_i_b, ffn1_w, ffn1_b, ffn2_w, ffn2_b):
    zrows = jnp.zeros((RPS, H), jnp.float32)
    src4, dst4 = _prep_edges(edge_index_left, edge_index_right)

    h0 = _h0(x_left, x_right, W_i1, W_i2)
    h = h0
    for _ in range(DEPTH - 1):
        agg = _sc_edge_agg(h, src4, dst4, zrows)
        h = _step(agg, h0, W_h1, W_h2)
    agg = _sc_edge_agg(h, src4, dst4, zrows)
    left_atom, right_atom = _last_step(agg, h0, W_h1, W_h2, x_left, x_right,
                                       W_o1, W_o2)

    left_out, right_out = _pool(left_atom, right_atom, batch_left, batch_right)

    left_scores, h_output = _coatt(left_atom, batch_left, right_out,
                                   w_i_w, w_i_b, prj_i_w, prj_i_b)

    noise = jax.random.uniform(jax.random.key(42), (B, H), jnp.float32)
    noise = noise / (jnp.linalg.norm(noise, axis=-1, keepdims=True) + 1e-12)

    right_scores, output, h_pert = _coatt_ffn(
        right_atom, batch_right, left_out, w_i_w, w_i_b, prj_i_w, prj_i_b,
        h_output, noise, ffn1_w, ffn1_b, ffn2_w, ffn2_b)
    return (output, h_output, h_pert, left_scores, right_scores,
            left_out, right_out)
